# baseline (speedup 1.0000x reference)
import functools

import jax
import jax.numpy as jnp
from jax import lax
from jax.experimental import pallas as pl
from jax.experimental.pallas import tpu as pltpu
from jax.experimental.pallas import tpu_sc as plsc

NC = 2
NS = 16
NW = NC * NS

VOC = 1000000
DIM = 32
NBLK = (VOC + 127) // 128
TROWS = VOC // 4
CH = 128


def _mesh():
    return plsc.VectorSubcoreMesh(core_axis_name="c", subcore_axis_name="s")


def _build_table_call():
    nblk_w = (NBLK - 1 + NW - 1) // NW

    @functools.partial(
        pl.kernel,
        mesh=_mesh(),
        out_type=jax.ShapeDtypeStruct((TROWS, 128), jnp.float32),
        scratch_types=[
            pltpu.VMEM((DIM, 128), jnp.float32),
            pltpu.VMEM((DIM, 128), jnp.float32),
        ],
        compiler_params=pltpu.CompilerParams(use_tc_tiling_on_sc=True,
                                             needs_layout_passes=False),
    )
    def k1(w_hbm, tail_hbm, tbl_hbm, src, dst):
        wid = lax.axis_index("s") * NC + lax.axis_index("c")
        iot = lax.iota(jnp.int32, 16)
        rows = (iot, iot + 16)

        @pl.when(wid == 0)
        def _():
            pltpu.sync_copy(tail_hbm, tbl_hbm.at[pl.ds(TROWS - 16, 16)])

        def body(i, carry):
            c = wid + i * NW

            @pl.when(c < NBLK - 1)
            def _():
                col0 = pl.multiple_of(c * 128, 128)
                pltpu.sync_copy(w_hbm.at[:, pl.ds(col0, 128)], src)
                for jr in range(32):
                    for half in range(8):
                        w0 = half * 16
                        colv = jnp.full((16,), jr * 4 + w0 // 32, jnp.int32)
                        val = plsc.load_gather(src, [rows[half % 2], colv])
                        dst[jr, pl.ds(w0, 16)] = val
                pltpu.sync_copy(dst, tbl_hbm.at[pl.ds(c * 32, 32)])
            return carry

        lax.fori_loop(0, nblk_w, body, 0)

    return k1


def _gather_call(b, h):
    b_per_w = b // NW
    n_bch = b_per_w // CH
    n_chunks = h * n_bch

    @functools.partial(
        pl.kernel,
        mesh=_mesh(),
        out_type=jax.ShapeDtypeStruct((h, DIM, b), jnp.float32),
        scratch_types=[
            pltpu.VMEM((CH,), jnp.int32),
            pltpu.VMEM((CH,), jnp.int32),
            pltpu.VMEM((CH, 128), jnp.float32),
            pltpu.VMEM((DIM, CH), jnp.float32),
            pltpu.SemaphoreType.DMA,
        ],
        compiler_params=pltpu.CompilerParams(use_tc_tiling_on_sc=True,
                                             needs_layout_passes=False),
    )
    def k2(ids_hbm, tbl_hbm, out_hbm, idxv, jbuf, gbuf, tbuf, sem):
        wid = lax.axis_index("s") * NC + lax.axis_index("c")
        b_base = wid * b_per_w
        iot = lax.iota(jnp.int32, 16)

        def body(k, carry):
            hh = k // n_bch
            b0 = b_base + (k % n_bch) * CH
            pltpu.sync_copy(ids_hbm.at[hh, pl.ds(b0, CH)], idxv)
            qs = []
            rows = []
            for g in range(8):
                xv = idxv[pl.ds(16 * g, 16)]
                jbuf[pl.ds(16 * g, 16)] = xv >> 2
                qs.append((xv & 3) * 32)
                rows.append(iot + 16 * g)
            pltpu.async_copy(tbl_hbm.at[jbuf], gbuf, sem).wait()
            for e in range(DIM):
                for g in range(8):
                    val = plsc.load_gather(gbuf, [rows[g], qs[g] + e])
                    tbuf[e, pl.ds(16 * g, 16)] = val
            pltpu.sync_copy(tbuf, out_hbm.at[hh, :, pl.ds(b0, CH)])
            return carry

        lax.fori_loop(0, n_chunks, body, 0)

    return k2


def kernel(input_ids, weight):
    b, h = input_ids.shape
    ids_t = input_ids.T
    w_t = weight.T
    tail = jnp.reshape(weight[VOC - 64:, :], (16, 128))
    tbl = _build_table_call()(w_t, tail)
    out_p = _gather_call(b, h)(ids_t, tbl)
    return jnp.transpose(out_p, (2, 0, 1))


# speedup vs baseline: 1.4743x; 1.4743x over previous
import functools

import jax
import jax.numpy as jnp
from jax import lax
from jax.experimental import pallas as pl
from jax.experimental.pallas import tpu as pltpu
from jax.experimental.pallas import tpu_sc as plsc

NC = 2
NS = 16
NW = NC * NS

VOC = 1000000
DIM = 32
NBLK = VOC // 128
TROWS = VOC // 4
GBLK = 2
GCOL = GBLK * 128
GROW = GBLK * 32
NG = NBLK // GBLK
K1_SLOTS = (NG + NW - 1) // NW
CH = 128

_params = pltpu.CompilerParams(use_tc_tiling_on_sc=True,
                               needs_layout_passes=False)


def _mesh():
    return plsc.VectorSubcoreMesh(core_axis_name="c", subcore_axis_name="s")


def _build_table_call():
    @functools.partial(
        pl.kernel,
        mesh=_mesh(),
        out_type=jax.ShapeDtypeStruct((TROWS, 128), jnp.float32),
        scratch_types=[
            pltpu.VMEM((2, DIM, GCOL), jnp.float32),
            pltpu.VMEM((2, GROW, 128), jnp.float32),
            pltpu.SemaphoreType.DMA((2,)),
            pltpu.SemaphoreType.DMA((2,)),
        ],
        compiler_params=_params,
    )
    def k1(w_hbm, tail_hbm, tbl_hbm, src, dst, ssem, wsem):
        wid = lax.axis_index("s") * NC + lax.axis_index("c")
        iot = lax.iota(jnp.int32, 16)
        rows = (iot, iot + 16)

        @pl.when(wid == 0)
        def _():
            pltpu.sync_copy(tail_hbm, tbl_hbm.at[pl.ds(TROWS - 16, 16)])

        def grp(s):
            return jnp.minimum(wid + s * NW, NG - 1)

        def src_copy(s, d):
            col0 = pl.multiple_of(grp(s) * GCOL, 128)
            return pltpu.make_async_copy(
                w_hbm.at[:, pl.ds(col0, GCOL)], src.at[d], ssem.at[d])

        def wb_copy(s, d):
            row0 = pl.multiple_of(grp(s) * GROW, 8)
            return pltpu.make_async_copy(
                dst.at[d], tbl_hbm.at[pl.ds(row0, GROW)], wsem.at[d])

        def compute(d):
            def jr_body(jr, carry):
                for b2 in range(GBLK):
                    for half in range(8):
                        w0 = half * 16
                        colv = jnp.full((16,), 0, jnp.int32) \
                            + (b2 * 128 + w0 // 32) + jr * 4
                        val = plsc.load_gather(
                            src.at[d], [rows[half % 2], colv])
                        dst[d, b2 * 32 + jr, pl.ds(w0, 16)] = val
                return carry

            lax.fori_loop(0, 32, jr_body, 0)

        def step(i, d, do_issue, do_comp, do_wbwait):
            if do_issue:
                src_copy(i, d).start()
            if do_comp:
                if do_wbwait:
                    wb_copy(i - 3, 1 - d).wait()
                src_copy(i - 1, 1 - d).wait()
                compute(1 - d)
                wb_copy(i - 1, 1 - d).start()

        for i in range(5):
            step(i, i % 2, i < K1_SLOTS, 0 <= i - 1 < K1_SLOTS, i - 3 >= 0)

        def loop_body(t, carry):
            step(2 * t + 5, 1, True, True, True)
            step(2 * t + 6, 0, True, True, True)
            return carry

        lax.fori_loop(0, (K1_SLOTS - 5) // 2, loop_body, 0)

        step(K1_SLOTS, K1_SLOTS % 2, False, True, True)
        wb_copy(K1_SLOTS - 2, (K1_SLOTS - 2) % 2).wait()
        wb_copy(K1_SLOTS - 1, (K1_SLOTS - 1) % 2).wait()

    return k1


def _gather_call(b, h):
    b_per_w = b // NW
    n_bch = b_per_w // CH
    n_chunks = h * n_bch

    @functools.partial(
        pl.kernel,
        mesh=_mesh(),
        out_type=jax.ShapeDtypeStruct((h, DIM, b), jnp.float32),
        scratch_types=[
            pltpu.VMEM((2, CH), jnp.int32),
            pltpu.VMEM((2, CH), jnp.int32),
            pltpu.VMEM((2, CH), jnp.int32),
            pltpu.VMEM((2, CH, 128), jnp.float32),
            pltpu.VMEM((2, DIM, CH), jnp.float32),
            pltpu.SemaphoreType.DMA((2,)),
            pltpu.SemaphoreType.DMA((2,)),
            pltpu.SemaphoreType.DMA((2,)),
        ],
        compiler_params=_params,
    )
    def k2(ids_hbm, tbl_hbm, out_hbm, idxv, jbuf, qbuf, gbuf, tbuf,
           isem, gsem, wsem):
        wid = lax.axis_index("s") * NC + lax.axis_index("c")
        b_base = wid * b_per_w
        iot = lax.iota(jnp.int32, 16)

        def hb(k):
            return k // n_bch, b_base + (k % n_bch) * CH

        def idx_copy(k, d):
            hh, b0 = hb(k)
            return pltpu.make_async_copy(
                ids_hbm.at[hh, pl.ds(b0, CH)], idxv.at[d], isem.at[d])

        def gather_copy(d):
            return pltpu.make_async_copy(
                tbl_hbm.at[jbuf.at[d]], gbuf.at[d], gsem.at[d])

        def wb_copy(k, d):
            hh, b0 = hb(k)
            return pltpu.make_async_copy(
                tbuf.at[d], out_hbm.at[hh, :, pl.ds(b0, CH)], wsem.at[d])

        def jcomp(d):
            for g in range(CH // 16):
                sl = pl.ds(16 * g, 16)
                xv = idxv[d, sl]
                jbuf[d, sl] = xv >> 2
                qbuf[d, sl] = (xv & 3) * 32

        def extract(d):
            qv = [qbuf[d, pl.ds(16 * g, 16)] for g in range(CH // 16)]
            rv = [iot + 16 * g for g in range(CH // 16)]

            def e_body(e, carry):
                for g in range(CH // 16):
                    val = plsc.load_gather(gbuf.at[d], [rv[g], qv[g] + e])
                    tbuf[d, e, pl.ds(16 * g, 16)] = val
                return carry

            lax.fori_loop(0, DIM, e_body, 0)

        def step(i, d, static=False):
            def active(k):
                return (0 <= k) and (k < n_chunks) if static else True

            if active(i - 1):
                idx_copy(i - 1, 1 - d).wait()
                jcomp(1 - d)
                gather_copy(1 - d).start()
            if active(i):
                idx_copy(i, d).start()
            if active(i - 2):
                if (not static) or i - 4 >= 0:
                    wb_copy(i - 4, d).wait()
                gather_copy(d).wait()
                extract(d)
                wb_copy(i - 2, d).start()

        for i in range(4):
            step(i, i % 2, static=True)

        def loop_body(t, carry):
            step(2 * t, 0)
            step(2 * t + 1, 1)
            return carry

        lax.fori_loop(2, n_chunks // 2, loop_body, 0)

        for i in (n_chunks, n_chunks + 1):
            step(i, i % 2, static=True)
        wb_copy(n_chunks - 2, 0).wait()
        wb_copy(n_chunks - 1, 1).wait()

    return k2


def kernel(input_ids, weight):
    b, h = input_ids.shape
    ids_t = input_ids.T
    w_t = weight.T
    tail = jnp.reshape(weight[VOC - 64:, :], (16, 128))
    tbl = _build_table_call()(w_t, tail)
    out_p = _gather_call(b, h)(ids_t, tbl)
    return jnp.transpose(out_p, (2, 0, 1))


# speedup vs baseline: 3.7688x; 2.5563x over previous
import functools

import jax
import jax.numpy as jnp
from jax import lax
from jax.experimental import pallas as pl
from jax.experimental.pallas import tpu as pltpu
from jax.experimental.pallas import tpu_sc as plsc

NC = 2
NS = 16
NW = NC * NS

VOC = 1000000
DIM = 32
NBLK = VOC // 128
TROWS = VOC // 4
GBLK = 2
GCOL = GBLK * 128
GROW = GBLK * 32
NG = NBLK // GBLK
K1_SLOTS = (NG + NW - 1) // NW
CH = 128

_params = pltpu.CompilerParams(use_tc_tiling_on_sc=True,
                               needs_layout_passes=False)


def _mesh():
    return plsc.VectorSubcoreMesh(core_axis_name="c", subcore_axis_name="s")


def _build_table_call():
    @functools.partial(
        pl.kernel,
        mesh=_mesh(),
        out_type=jax.ShapeDtypeStruct((TROWS, 128), jnp.float32),
        scratch_types=[
            pltpu.VMEM((2, DIM, GCOL), jnp.float32),
            pltpu.VMEM((2, GROW, 128), jnp.float32),
            pltpu.SemaphoreType.DMA((2,)),
            pltpu.SemaphoreType.DMA((2,)),
        ],
        compiler_params=_params,
    )
    def k1(w_hbm, tail_hbm, tbl_hbm, src, dst, ssem, wsem):
        wid = lax.axis_index("s") * NC + lax.axis_index("c")
        iot = lax.iota(jnp.int32, 16)
        rows = (iot, iot + 16)

        @pl.when(wid == 0)
        def _():
            pltpu.sync_copy(tail_hbm, tbl_hbm.at[pl.ds(TROWS - 16, 16)])

        def grp(s):
            return jnp.minimum(wid + s * NW, NG - 1)

        def src_copy(s, d):
            col0 = pl.multiple_of(grp(s) * GCOL, 128)
            return pltpu.make_async_copy(
                w_hbm.at[:, pl.ds(col0, GCOL)], src.at[d], ssem.at[d])

        def wb_copy(s, d):
            row0 = pl.multiple_of(grp(s) * GROW, 8)
            return pltpu.make_async_copy(
                dst.at[d], tbl_hbm.at[pl.ds(row0, GROW)], wsem.at[d])

        civ = [iot + 4 * jr0 for jr0 in range(0, 32, 4)]
        drv = [jr0 + iot // 4 for jr0 in range(0, 32, 4)]
        dcb = (iot % 4) * 32

        def compute(d):
            def e_body(e0, carry):
                ev = (iot + e0) & 31
                dcv = dcb + ev
                for b2 in range(GBLK):
                    for q8 in range(8):
                        val = plsc.load_gather(
                            src.at[d], [ev, civ[q8] + b2 * 128])
                        plsc.store_scatter(
                            dst.at[d], [drv[q8] + b2 * 32, dcv], val)
                return carry

            lax.fori_loop(0, 32, e_body, 0)

        def step(i, d, do_issue, do_comp, do_wbwait):
            if do_issue:
                src_copy(i, d).start()
            if do_comp:
                if do_wbwait:
                    wb_copy(i - 3, 1 - d).wait()
                src_copy(i - 1, 1 - d).wait()
                compute(1 - d)
                wb_copy(i - 1, 1 - d).start()

        for i in range(5):
            step(i, i % 2, i < K1_SLOTS, 0 <= i - 1 < K1_SLOTS, i - 3 >= 0)

        def loop_body(t, carry):
            step(2 * t + 5, 1, True, True, True)
            step(2 * t + 6, 0, True, True, True)
            return carry

        lax.fori_loop(0, (K1_SLOTS - 5) // 2, loop_body, 0)

        step(K1_SLOTS, K1_SLOTS % 2, False, True, True)
        wb_copy(K1_SLOTS - 2, (K1_SLOTS - 2) % 2).wait()
        wb_copy(K1_SLOTS - 1, (K1_SLOTS - 1) % 2).wait()

    return k1


def _gather_call(b, h):
    b_per_w = b // NW
    n_bch = b_per_w // CH
    n_chunks = h * n_bch

    @functools.partial(
        pl.kernel,
        mesh=_mesh(),
        out_type=jax.ShapeDtypeStruct((h, DIM, b), jnp.float32),
        scratch_types=[
            pltpu.VMEM((2, CH), jnp.int32),
            pltpu.VMEM((2, CH), jnp.int32),
            pltpu.VMEM((2, CH), jnp.int32),
            pltpu.VMEM((2, CH, 128), jnp.float32),
            pltpu.VMEM((2, DIM, CH), jnp.float32),
            pltpu.SemaphoreType.DMA((2,)),
            pltpu.SemaphoreType.DMA((2,)),
            pltpu.SemaphoreType.DMA((2,)),
        ],
        compiler_params=_params,
    )
    def k2(ids_hbm, tbl_hbm, out_hbm, idxv, jbuf, qbuf, gbuf, tbuf,
           isem, gsem, wsem):
        wid = lax.axis_index("s") * NC + lax.axis_index("c")
        b_base = wid * b_per_w
        iot = lax.iota(jnp.int32, 16)

        def hb(k):
            return k // n_bch, b_base + (k % n_bch) * CH

        def idx_copy(k, d):
            hh, b0 = hb(k)
            return pltpu.make_async_copy(
                ids_hbm.at[hh, pl.ds(b0, CH)], idxv.at[d], isem.at[d])

        def gather_copy(d):
            return pltpu.make_async_copy(
                tbl_hbm.at[jbuf.at[d]], gbuf.at[d], gsem.at[d])

        def wb_copy(k, d):
            hh, b0 = hb(k)
            return pltpu.make_async_copy(
                tbuf.at[d], out_hbm.at[hh, :, pl.ds(b0, CH)], wsem.at[d])

        def jcomp(d):
            for g in range(CH // 16):
                sl = pl.ds(16 * g, 16)
                xv = idxv[d, sl]
                jbuf[d, sl] = xv >> 2
                qbuf[d, sl] = (xv & 3) * 32

        def extract(d):
            qv = [qbuf[d, pl.ds(16 * g, 16)] for g in range(CH // 16)]
            rv = [iot + 16 * g for g in range(CH // 16)]

            def e_body(e0, carry):
                ev = (iot + e0) & 31
                for g in range(CH // 16):
                    val = plsc.load_gather(gbuf.at[d], [rv[g], qv[g] + ev])
                    plsc.store_scatter(tbuf.at[d], [ev, rv[g]], val)
                return carry

            lax.fori_loop(0, DIM, e_body, 0)

        def step(i, d, static=False):
            def active(k):
                return (0 <= k) and (k < n_chunks) if static else True

            if active(i - 1):
                idx_copy(i - 1, 1 - d).wait()
                jcomp(1 - d)
                gather_copy(1 - d).start()
            if active(i):
                idx_copy(i, d).start()
            if active(i - 2):
                if (not static) or i - 4 >= 0:
                    wb_copy(i - 4, d).wait()
                gather_copy(d).wait()
                extract(d)
                wb_copy(i - 2, d).start()

        for i in range(4):
            step(i, i % 2, static=True)

        def loop_body(t, carry):
            step(2 * t, 0)
            step(2 * t + 1, 1)
            return carry

        lax.fori_loop(2, n_chunks // 2, loop_body, 0)

        for i in (n_chunks, n_chunks + 1):
            step(i, i % 2, static=True)
        wb_copy(n_chunks - 2, 0).wait()
        wb_copy(n_chunks - 1, 1).wait()

    return k2


def kernel(input_ids, weight):
    b, h = input_ids.shape
    ids_t = input_ids.T
    w_t = weight.T
    tail = jnp.reshape(weight[VOC - 64:, :], (16, 128))
    tbl = _build_table_call()(w_t, tail)
    out_p = _gather_call(b, h)(ids_t, tbl)
    return jnp.transpose(out_p, (2, 0, 1))
